# baseline (speedup 1.0000x reference)
import jax
import jax.numpy as jnp
from jax.experimental import pallas as pl
from jax.experimental.pallas import tpu as pltpu


def _copy_kernel(x_ref, o_ref, sem):
    cp = pltpu.make_async_copy(x_ref, o_ref, sem)
    cp.start()
    cp.wait()


def kernel(x):
    return pl.pallas_call(
        _copy_kernel,
        out_shape=jax.ShapeDtypeStruct(x.shape, x.dtype),
        in_specs=[pl.BlockSpec(memory_space=pl.ANY)],
        out_specs=pl.BlockSpec(memory_space=pl.ANY),
        scratch_shapes=[pltpu.SemaphoreType.DMA],
    )(x)


# speedup vs baseline: 49.0839x; 49.0839x over previous
import jax
import jax.numpy as jnp
from jax.experimental import pallas as pl
from jax.experimental.pallas import tpu as pltpu

_BLOCK0 = 8


def _copy_kernel(x_ref, o_ref):
    o_ref[...] = x_ref[...]


def kernel(x):
    b, c, f = x.shape
    return pl.pallas_call(
        _copy_kernel,
        out_shape=jax.ShapeDtypeStruct(x.shape, x.dtype),
        grid=(b // _BLOCK0,),
        in_specs=[pl.BlockSpec((_BLOCK0, c, f), lambda i: (i, 0, 0))],
        out_specs=pl.BlockSpec((_BLOCK0, c, f), lambda i: (i, 0, 0)),
    )(x)
